# baseline (speedup 1.0000x reference)
import functools

import jax
import jax.numpy as jnp
from jax.experimental import pallas as pl
from jax.experimental.pallas import tpu as pltpu

N_NODES = 10000
F_DIM = 128
H_DIM = 256
E_DIM = 256
NBR = 160000
HALF = NBR // 2

RC1 = 3200
NA = HALF // RC1
RC2 = 2000
NB = HALF // RC2
NT = 2000
N_TILE = N_NODES // NT
NTB = 1000
N_TILE_B = N_NODES // NTB
I_C = NB
I_D = I_C + N_TILE
N_STEPS = I_D + N_TILE


def _fused_body(n1a_ref, n1b_ref, n2a_ref, n2b_ref, nf_ref,
                Ws1_ref, bs1_ref, Wn1_ref, bn1_ref, g1_ref, be1_ref,
                Ws2_ref, bs2_ref, Wn2_ref, bn2_ref, g2_ref, be2_ref,
                out_ref, z_scr, s1_ref, s2_ref, css_ref):
    i = pl.program_id(0)

    @pl.when(i == 0)
    def _init():
        s1_ref[...] = jnp.zeros_like(s1_ref)
        s2_ref[...] = jnp.zeros_like(s2_ref)
        css_ref[...] = jnp.zeros_like(css_ref)

    @pl.when(i < NA)
    def _reduce_n1():
        s1_ref[...] += (jnp.sum(n1a_ref[...], axis=0, keepdims=True)
                        + jnp.sum(n1b_ref[...], axis=0, keepdims=True))

    @pl.when(i < NB)
    def _reduce_n2():
        s2_ref[...] += (jnp.sum(n2a_ref[...], axis=0, keepdims=True)
                        + jnp.sum(n2b_ref[...], axis=0, keepdims=True))

    @pl.when((i < 5 * N_TILE) & (i % 5 == 0))
    def _mm_z1():
        t = i // 5
        z_scr[pl.ds(t * NT, NT), :] = jnp.dot(
            nf_ref[...], Ws1_ref[...], preferred_element_type=jnp.float32)

    @pl.when((i >= NA) & (i < NA + N_TILE_B))
    def _mm_h1z2():
        t = i - NA
        inv_nbr = jnp.float32(1.0 / NBR)
        agg1 = s1_ref[...] * inv_nbr
        row1 = jnp.dot(agg1, Wn1_ref[...], preferred_element_type=jnp.float32)
        row1 = row1 + bn1_ref[...] + bs1_ref[...]
        o1 = z_scr[pl.ds(t * NTB, NTB), :] + row1
        mu = jnp.mean(o1, axis=-1, keepdims=True)
        xc = o1 - mu
        var = jnp.mean(xc * xc, axis=-1, keepdims=True)
        h1 = xc * jax.lax.rsqrt(var + 1e-5) * g1_ref[...] + be1_ref[...]
        h1 = jnp.maximum(h1, 0.0)
        z_scr[pl.ds(t * NTB, NTB), :] = jnp.dot(
            h1, Ws2_ref[...], preferred_element_type=jnp.float32)

    @pl.when((i >= I_C) & (i < I_D))
    def _phase_c():
        t = i - I_C
        inv_nbr = jnp.float32(1.0 / NBR)
        agg2 = s2_ref[...] * inv_nbr
        row2 = jnp.dot(agg2, Wn2_ref[...], preferred_element_type=jnp.float32)
        row2 = row2 + bn2_ref[...] + bs2_ref[...]
        o2 = z_scr[pl.ds(t * NT, NT), :] + row2
        mu2 = jnp.mean(o2, axis=-1, keepdims=True)
        xc2 = o2 - mu2
        var2 = jnp.mean(xc2 * xc2, axis=-1, keepdims=True)
        h2 = xc2 * jax.lax.rsqrt(var2 + 1e-5) * g2_ref[...] + be2_ref[...]
        h2 = jnp.maximum(h2, 0.0)
        z_scr[pl.ds(t * NT, NT), :] = h2
        css_ref[...] += jnp.sum(h2 * h2, axis=0, keepdims=True)

    @pl.when(i >= I_D)
    def _phase_d():
        t = i - I_D
        inv = 1.0 / jnp.maximum(jnp.sqrt(css_ref[...]), 1e-12)
        out_ref[...] = z_scr[pl.ds(t * NT, NT), :] * inv


@jax.jit
def _run(node_feat, n1, n2, Ws1, bs1, Wn1, bn1, g1, be1,
         Ws2, bs2, Wn2, bn2, g2, be2):
    row = lambda v: v.reshape(1, -1)
    full = lambda a: pl.BlockSpec(a.shape, lambda i: (0,) * a.ndim)
    weights = [Ws1, row(bs1), Wn1, row(bn1), row(g1), row(be1),
               Ws2, row(bs2), Wn2, row(bn2), row(g2), row(be2)]

    out = pl.pallas_call(
        _fused_body,
        grid=(N_STEPS,),
        in_specs=[
            pl.BlockSpec((RC1, F_DIM),
                         lambda i: (jnp.minimum(i, NA - 1), 0)),
            pl.BlockSpec((RC1, F_DIM),
                         lambda i: (NA + jnp.minimum(i, NA - 1), 0)),
            pl.BlockSpec((RC2, H_DIM),
                         lambda i: (jnp.minimum(i, NB - 1), 0)),
            pl.BlockSpec((RC2, H_DIM),
                         lambda i: (NB + jnp.minimum(i, NB - 1), 0)),
            pl.BlockSpec((NT, F_DIM),
                         lambda i: (jnp.minimum(i // 5, N_TILE - 1), 0)),
        ] + [full(w) for w in weights],
        out_specs=pl.BlockSpec((NT, E_DIM),
                               lambda i: (jnp.clip(i - I_D, 0, N_TILE - 1), 0)),
        out_shape=jax.ShapeDtypeStruct((N_NODES, E_DIM), jnp.float32),
        scratch_shapes=[
            pltpu.VMEM((N_NODES, H_DIM), jnp.float32),
            pltpu.VMEM((1, F_DIM), jnp.float32),
            pltpu.VMEM((1, H_DIM), jnp.float32),
            pltpu.VMEM((1, E_DIM), jnp.float32),
        ],
        compiler_params=pltpu.CompilerParams(
            dimension_semantics=("arbitrary",),
        ),
    )(n1, n1, n2, n2, node_feat, *weights)
    return out


def kernel(node_feat, neighbor_feats_l1, neighbor_feats_l2,
           W_self1, b_self1, W_nbr1, b_nbr1, g1, be1,
           W_self2, b_self2, W_nbr2, b_nbr2, g2, be2):
    return _run(node_feat, neighbor_feats_l1, neighbor_feats_l2,
                W_self1, b_self1, W_nbr1, b_nbr1, g1, be1,
                W_self2, b_self2, W_nbr2, b_nbr2, g2, be2)


# speedup vs baseline: 1.0596x; 1.0596x over previous
import functools

import jax
import jax.numpy as jnp
from jax.experimental import pallas as pl
from jax.experimental.pallas import tpu as pltpu

N_NODES = 10000
F_DIM = 128
H_DIM = 256
E_DIM = 256
NBR = 160000
HALF = NBR // 2

RC = 2000
N_RED = HALF // RC
NT = 5000
N_TILE = N_NODES // NT


def _reduce_body(n1a_ref, n1b_ref, n2a_ref, n2b_ref, s1_ref, s2_ref):
    i = pl.program_id(0)

    @pl.when(i == 0)
    def _():
        s1_ref[...] = jnp.zeros_like(s1_ref)
        s2_ref[...] = jnp.zeros_like(s2_ref)

    s1_ref[...] += (jnp.sum(n1a_ref[...], axis=0, keepdims=True)
                    + jnp.sum(n1b_ref[...], axis=0, keepdims=True))
    s2_ref[...] += (jnp.sum(n2a_ref[...], axis=0, keepdims=True)
                    + jnp.sum(n2b_ref[...], axis=0, keepdims=True))


def _layer_norm_relu(x, g, b):
    mu = jnp.mean(x, axis=-1, keepdims=True)
    ms = jnp.mean(x * x, axis=-1, keepdims=True)
    var = ms - mu * mu
    y = (x - mu) * jax.lax.rsqrt(var + 1e-5) * g + b
    return jnp.maximum(y, 0.0)


def _dense_body(nf_ref, s1_ref, s2_ref,
                Ws1_ref, bs1_ref, Wn1_ref, bn1_ref, g1_ref, be1_ref,
                Ws2_ref, bs2_ref, Wn2_ref, bn2_ref, g2_ref, be2_ref,
                out_ref, h2_scr, css_ref):
    i = pl.program_id(0)
    t = i % N_TILE

    @pl.when(i == 0)
    def _():
        css_ref[...] = jnp.zeros_like(css_ref)

    @pl.when(i < N_TILE)
    def _compute():
        inv_nbr = jnp.float32(1.0 / NBR)
        agg1 = s1_ref[...] * inv_nbr
        row1 = jnp.dot(agg1, Wn1_ref[...], preferred_element_type=jnp.float32)
        row1 = row1 + bn1_ref[...] + bs1_ref[...]

        x = nf_ref[...]
        out1 = jnp.dot(x, Ws1_ref[...], preferred_element_type=jnp.float32)
        h1 = _layer_norm_relu(out1 + row1, g1_ref[...], be1_ref[...])

        agg2 = s2_ref[...] * inv_nbr
        row2 = jnp.dot(agg2, Wn2_ref[...], preferred_element_type=jnp.float32)
        row2 = row2 + bn2_ref[...] + bs2_ref[...]
        out2 = jnp.dot(h1, Ws2_ref[...], preferred_element_type=jnp.float32)
        h2 = _layer_norm_relu(out2 + row2, g2_ref[...], be2_ref[...])

        h2_scr[pl.ds(t * NT, NT), :] = h2
        css_ref[...] += jnp.sum(h2 * h2, axis=0, keepdims=True)

    @pl.when(i >= N_TILE)
    def _normalize():
        norm = jnp.sqrt(css_ref[...])
        inv = 1.0 / jnp.maximum(norm, 1e-12)
        out_ref[...] = h2_scr[pl.ds(t * NT, NT), :] * inv


@jax.jit
def _run(node_feat, n1, n2, Ws1, bs1, Wn1, bn1, g1, be1,
         Ws2, bs2, Wn2, bn2, g2, be2):
    sums = pl.pallas_call(
        _reduce_body,
        grid=(N_RED,),
        in_specs=[
            pl.BlockSpec((RC, F_DIM), lambda i: (i, 0)),
            pl.BlockSpec((RC, F_DIM), lambda i: (N_RED + i, 0)),
            pl.BlockSpec((RC, H_DIM), lambda i: (i, 0)),
            pl.BlockSpec((RC, H_DIM), lambda i: (N_RED + i, 0)),
        ],
        out_specs=[
            pl.BlockSpec((1, F_DIM), lambda i: (0, 0)),
            pl.BlockSpec((1, H_DIM), lambda i: (0, 0)),
        ],
        out_shape=[
            jax.ShapeDtypeStruct((1, F_DIM), jnp.float32),
            jax.ShapeDtypeStruct((1, H_DIM), jnp.float32),
        ],
        compiler_params=pltpu.CompilerParams(
            dimension_semantics=("arbitrary",),
        ),
    )(n1, n1, n2, n2)
    s1, s2 = sums

    row = lambda v: v.reshape(1, -1)
    full = lambda a: pl.BlockSpec(a.shape, lambda i: (0,) * a.ndim)
    weights = [Ws1, row(bs1), Wn1, row(bn1), row(g1), row(be1),
               Ws2, row(bs2), Wn2, row(bn2), row(g2), row(be2)]

    out = pl.pallas_call(
        _dense_body,
        grid=(2 * N_TILE,),
        in_specs=[
            pl.BlockSpec((NT, F_DIM), lambda i: (jnp.minimum(i, N_TILE - 1), 0)),
            full(s1), full(s2),
        ] + [full(w) for w in weights],
        out_specs=pl.BlockSpec((NT, E_DIM),
                               lambda i: (jnp.maximum(i - N_TILE, 0), 0)),
        out_shape=jax.ShapeDtypeStruct((N_NODES, E_DIM), jnp.float32),
        scratch_shapes=[
            pltpu.VMEM((N_NODES, E_DIM), jnp.float32),
            pltpu.VMEM((1, E_DIM), jnp.float32),
        ],
        compiler_params=pltpu.CompilerParams(
            dimension_semantics=("arbitrary",),
        ),
    )(node_feat, s1, s2, *weights)
    return out


def kernel(node_feat, neighbor_feats_l1, neighbor_feats_l2,
           W_self1, b_self1, W_nbr1, b_nbr1, g1, be1,
           W_self2, b_self2, W_nbr2, b_nbr2, g2, be2):
    return _run(node_feat, neighbor_feats_l1, neighbor_feats_l2,
                W_self1, b_self1, W_nbr1, b_nbr1, g1, be1,
                W_self2, b_self2, W_nbr2, b_nbr2, g2, be2)
